# Initial kernel scaffold; baseline (speedup 1.0000x reference)
import jax
import jax.numpy as jnp
from jax.experimental import pallas as pl


def kernel(x, pos, sigma_att):
    raise NotImplementedError("write your pallas kernel here")


# speedup vs baseline: 7.3944x; 7.3944x over previous
import functools

import jax
import jax.numpy as jnp
from jax.experimental import pallas as pl

H = 8
M = 64
TOPK = 2
NM = 2048
N = NM // M
E = 256
EH = E // H
SCALE = float(E) ** -0.5
BQ = 256


def _attn_kernel(xh_ref, out_ref):
    k = xh_ref[0]
    qi = pl.program_id(1)
    q = xh_ref[0, pl.ds(qi * BQ, BQ), :]

    s = jax.lax.dot_general(q, k, (((1,), (1,)), ((), ())),
                            preferred_element_type=jnp.float32) * SCALE

    means = jnp.mean(k.reshape(N, M, EH), axis=1)
    r = jax.lax.dot_general(q, means, (((1,), (1,)), ((), ())),
                            preferred_element_type=jnp.float32) * SCALE

    p = jax.nn.softmax(r, axis=-1)

    i1 = jnp.argmax(p, axis=-1)
    v1 = jnp.max(p, axis=-1)
    ball_iota = jax.lax.broadcasted_iota(jnp.int32, (BQ, N), 1)
    p2 = jnp.where(ball_iota == i1[:, None], -jnp.inf, p)
    i2 = jnp.argmax(p2, axis=-1)
    v2 = jnp.max(p2, axis=-1)

    key_ball = jax.lax.broadcasted_iota(jnp.int32, (BQ, NM), 1) // M
    sel = ((key_ball == i1[:, None]) & (v1[:, None] > 1e-10)) | (
        (key_ball == i2[:, None]) & (v2[:, None] > 1e-10))

    neg = -jnp.finfo(jnp.float32).max
    s = jnp.where(sel, s, neg)

    smax = jnp.max(s, axis=-1, keepdims=True)
    e = jnp.exp(s - smax)
    attn = e / jnp.sum(e, axis=-1, keepdims=True)

    out = jax.lax.dot_general(attn, k, (((1,), (0,)), ((), ())),
                              preferred_element_type=jnp.float32)
    out_ref[0, pl.ds(qi * BQ, BQ), :] = out


@jax.jit
def _run(x):
    xh = jnp.transpose(x.reshape(NM, H, EH), (1, 0, 2))

    out = pl.pallas_call(
        _attn_kernel,
        grid=(H, NM // BQ),
        in_specs=[
            pl.BlockSpec((1, NM, EH), lambda h, i: (h, 0, 0)),
        ],
        out_specs=pl.BlockSpec((1, NM, EH), lambda h, i: (h, 0, 0)),
        out_shape=jax.ShapeDtypeStruct((H, NM, EH), jnp.float32),
    )(xh)
    return jnp.transpose(out, (1, 0, 2)).reshape(NM, E)


def kernel(x, pos, sigma_att):
    return _run(x)


# speedup vs baseline: 8.5395x; 1.1549x over previous
import functools

import jax
import jax.numpy as jnp
from jax.experimental import pallas as pl

H = 8
M = 64
TOPK = 2
NM = 2048
N = NM // M
E = 256
EH = E // H
SCALE = float(E) ** -0.5
BQ = 256


def _attn_kernel(xh_ref, out_ref):
    k = xh_ref[0]
    qi = pl.program_id(1)
    q = xh_ref[0, pl.ds(qi * BQ, BQ), :]

    qb = q.astype(jnp.bfloat16)
    kb = k.astype(jnp.bfloat16)
    s = jax.lax.dot_general(qb, kb, (((1,), (1,)), ((), ())),
                            preferred_element_type=jnp.float32) * SCALE

    means = jnp.mean(k.reshape(N, M, EH), axis=1)
    r = jax.lax.dot_general(q, means, (((1,), (1,)), ((), ())),
                            preferred_element_type=jnp.float32) * SCALE

    p = jax.nn.softmax(r, axis=-1)

    i1 = jnp.argmax(p, axis=-1)
    v1 = jnp.max(p, axis=-1)
    ball_iota = jax.lax.broadcasted_iota(jnp.int32, (BQ, N), 1)
    p2 = jnp.where(ball_iota == i1[:, None], -jnp.inf, p)
    i2 = jnp.argmax(p2, axis=-1)
    v2 = jnp.max(p2, axis=-1)

    key_ball = jax.lax.broadcasted_iota(jnp.int32, (BQ, NM), 1) // M
    sel = ((key_ball == i1[:, None]) & (v1[:, None] > 1e-10)) | (
        (key_ball == i2[:, None]) & (v2[:, None] > 1e-10))

    neg = -jnp.finfo(jnp.float32).max
    s = jnp.where(sel, s, neg)

    smax = jnp.max(s, axis=-1, keepdims=True)
    e = jnp.exp(s - smax)
    attn = e / jnp.sum(e, axis=-1, keepdims=True)

    out = jax.lax.dot_general(attn.astype(jnp.bfloat16), kb,
                              (((1,), (0,)), ((), ())),
                              preferred_element_type=jnp.float32)
    out_ref[0, pl.ds(qi * BQ, BQ), :] = out


@jax.jit
def _run(x):
    xh = jnp.transpose(x.reshape(NM, H, EH), (1, 0, 2))

    out = pl.pallas_call(
        _attn_kernel,
        grid=(H, NM // BQ),
        in_specs=[
            pl.BlockSpec((1, NM, EH), lambda h, i: (h, 0, 0)),
        ],
        out_specs=pl.BlockSpec((1, NM, EH), lambda h, i: (h, 0, 0)),
        out_shape=jax.ShapeDtypeStruct((H, NM, EH), jnp.float32),
    )(xh)
    return jnp.transpose(out, (1, 0, 2)).reshape(NM, E)


def kernel(x, pos, sigma_att):
    return _run(x)


# speedup vs baseline: 10.3316x; 1.2099x over previous
import functools

import jax
import jax.numpy as jnp
from jax.experimental import pallas as pl

H = 8
M = 64
TOPK = 2
NM = 2048
N = NM // M
E = 256
EH = E // H
SCALE = float(E) ** -0.5
BQ = 256


def _attn_kernel(xh_ref, onehot_ref, out_ref):
    k = xh_ref[0]
    qi = pl.program_id(1)
    q = xh_ref[0, pl.ds(qi * BQ, BQ), :]

    qb = q.astype(jnp.bfloat16)
    kb = k.astype(jnp.bfloat16)
    s = jax.lax.dot_general(qb, kb, (((1,), (1,)), ((), ())),
                            preferred_element_type=jnp.float32) * SCALE

    means = jnp.mean(k.reshape(N, M, EH), axis=1)
    r = jax.lax.dot_general(q, means, (((1,), (1,)), ((), ())),
                            preferred_element_type=jnp.float32) * SCALE

    p = jax.nn.softmax(r, axis=-1)

    i1 = jnp.argmax(p, axis=-1)
    v1 = jnp.max(p, axis=-1)
    ball_iota = jax.lax.broadcasted_iota(jnp.int32, (BQ, N), 1)
    p2 = jnp.where(ball_iota == i1[:, None], -jnp.inf, p)
    i2 = jnp.argmax(p2, axis=-1)
    v2 = jnp.max(p2, axis=-1)

    sel = ((ball_iota == i1[:, None]) & (v1[:, None] > 1e-10)) | (
        (ball_iota == i2[:, None]) & (v2[:, None] > 1e-10))

    negb = jnp.where(sel, 0.0, -3.0e38).astype(jnp.bfloat16)
    bias = jax.lax.dot_general(negb, onehot_ref[...], (((1,), (0,)), ((), ())),
                               preferred_element_type=jnp.float32)
    s = s + bias

    smax = jnp.max(s, axis=-1, keepdims=True)
    e = jnp.exp(s - smax)
    denom = jnp.sum(e, axis=-1, keepdims=True)

    out = jax.lax.dot_general(e.astype(jnp.bfloat16), kb,
                              (((1,), (0,)), ((), ())),
                              preferred_element_type=jnp.float32)
    out_ref[0, pl.ds(qi * BQ, BQ), :] = out * (1.0 / denom)


@jax.jit
def _run(x):
    xh = jnp.transpose(x.reshape(NM, H, EH), (1, 0, 2))
    onehot = (jnp.arange(NM, dtype=jnp.int32)[None, :] // M ==
              jnp.arange(N, dtype=jnp.int32)[:, None]).astype(jnp.bfloat16)

    out = pl.pallas_call(
        _attn_kernel,
        grid=(H, NM // BQ),
        in_specs=[
            pl.BlockSpec((1, NM, EH), lambda h, i: (h, 0, 0)),
            pl.BlockSpec((N, NM), lambda h, i: (0, 0)),
        ],
        out_specs=pl.BlockSpec((1, NM, EH), lambda h, i: (h, 0, 0)),
        out_shape=jax.ShapeDtypeStruct((H, NM, EH), jnp.float32),
    )(xh, onehot)
    return jnp.transpose(out, (1, 0, 2)).reshape(NM, E)


def kernel(x, pos, sigma_att):
    return _run(x)
